# baseline (speedup 1.0000x reference)
import jax
import jax.numpy as jnp
from jax import lax
from jax.experimental import pallas as pl
from jax.experimental.pallas import tpu as pltpu
from jax.experimental.pallas import tpu_sc as plsc

N = 10000
D = 128
E = 320000
DH = D // 2

NC = 2
NS = 16
NW = NC * NS
NPAD = 10240
EPAD = 327680
K = 128
ROWS_A = EPAD // K // NW
ROWS_C = EPAD // K // NS
RPT = NPAD // NS

_MESH = plsc.VectorSubcoreMesh(
    core_axis_name="c", subcore_axis_name="s", num_cores=NC, num_subcores=NS
)


def _deg_body(dst2_hbm, degp_hbm, didx_v, ones_v, zrow_v, hist_sh, asem):
    c = lax.axis_index("c")
    s = lax.axis_index("s")
    _ZV = jnp.zeros((16,), jnp.float32)
    _OV = jnp.ones((16,), jnp.float32)

    def fill_o(i, carry):
        ones_v[pl.ds(i * 16, 16)] = _OV
        return carry

    lax.fori_loop(0, K // 16, fill_o, 0)

    def fill_z(i, carry):
        zrow_v[pl.ds(i * 16, 16)] = _ZV
        return carry

    lax.fori_loop(0, RPT // 16, fill_z, 0)

    my_rows = pl.multiple_of(s * RPT, 8)
    pltpu.sync_copy(zrow_v, hist_sh.at[pl.ds(my_rows, RPT)])
    row0 = pl.multiple_of((c * NS + s) * ROWS_A, 8)
    pltpu.sync_copy(dst2_hbm.at[pl.ds(row0, ROWS_A)], didx_v)
    plsc.subcore_barrier()

    def fire(k, carry):
        pltpu.async_copy(ones_v, hist_sh.at[didx_v.at[k]], asem, add=True)
        return carry

    lax.fori_loop(0, ROWS_A, fire, 0)

    def drain(k, carry):
        pltpu.make_async_copy(ones_v, hist_sh.at[didx_v.at[0]], asem).wait()
        return carry

    lax.fori_loop(0, ROWS_A, drain, 0)
    plsc.subcore_barrier()
    pltpu.sync_copy(hist_sh.at[pl.ds(my_rows, RPT)],
                    degp_hbm.at[c, pl.ds(my_rows, RPT)])


_deg_kernel = pl.kernel(
    _deg_body,
    out_type=jax.ShapeDtypeStruct((NC, NPAD), jnp.float32),
    mesh=_MESH,
    scratch_types=[
        pltpu.VMEM((ROWS_A, K), jnp.int32),
        pltpu.VMEM((K,), jnp.float32),
        pltpu.VMEM((RPT,), jnp.float32),
        pltpu.VMEM_SHARED((NPAD,), jnp.float32),
        pltpu.SemaphoreType.DMA,
    ],
)


NBUF = 2
G = ROWS_A // NBUF

def _agg_body(packed_hbm, y_hbm, aggp_hbm, packed_v, sidx_v, didx_v, rows_v,
              agg_sh, *sems):
    gsem = sems[:NBUF]
    ssem = sems[NBUF:]
    c = lax.axis_index("c")
    s = lax.axis_index("s")
    _ZV = jnp.zeros((16,), jnp.float32)

    def fz(i, carry):
        def fz2(j, carry2):
            rows_v[0, i, pl.ds(j * 16, 16)] = _ZV
            return carry2
        return lax.fori_loop(0, D // 16, fz2, carry)

    lax.fori_loop(0, K, fz, 0)

    my_rows = pl.multiple_of(s * RPT, 8)

    def fzd(j, carry):
        off = pl.multiple_of(my_rows + j * K, 8)
        pltpu.sync_copy(rows_v.at[0], agg_sh.at[pl.ds(off, K)])
        return carry

    lax.fori_loop(0, RPT // K, fzd, 0)

    row0 = pl.multiple_of((c * NS + s) * ROWS_A, 8)
    pltpu.sync_copy(packed_hbm.at[pl.ds(row0, ROWS_A)], packed_v)
    plsc.subcore_barrier()

    def unpack(k, b):
        def up(j, carry):
            p = packed_v[k, pl.ds(j * 16, 16)]
            sidx_v[b, pl.ds(j * 16, 16)] = p & 0xFFFF
            didx_v[b, pl.ds(j * 16, 16)] = lax.shift_right_logical(p, 16)
            return carry
        lax.fori_loop(0, K // 16, up, 0)

    for b in range(NBUF):
        unpack(b, b)
        pltpu.async_copy(y_hbm.at[sidx_v.at[b]], rows_v.at[b], gsem[b])

    def outer(g, carry):
        for b in range(NBUF):
            k = g * NBUF + b
            pltpu.make_async_copy(y_hbm.at[sidx_v.at[b]], rows_v.at[b],
                                  gsem[b]).wait()
            pltpu.async_copy(rows_v.at[b], agg_sh.at[didx_v.at[b]], ssem[b],
                             add=True)
            pltpu.make_async_copy(rows_v.at[b], agg_sh.at[didx_v.at[b]],
                                  ssem[b]).wait()
            unpack(k + NBUF, b)
            pltpu.async_copy(y_hbm.at[sidx_v.at[b]], rows_v.at[b], gsem[b])
        return carry

    lax.fori_loop(0, G - 1, outer, 0)
    for b in range(NBUF):
        pltpu.make_async_copy(y_hbm.at[sidx_v.at[b]], rows_v.at[b],
                              gsem[b]).wait()
        pltpu.async_copy(rows_v.at[b], agg_sh.at[didx_v.at[b]], ssem[b],
                         add=True)
    for b in range(NBUF):
        pltpu.make_async_copy(rows_v.at[b], agg_sh.at[didx_v.at[b]],
                              ssem[b]).wait()

    plsc.subcore_barrier()
    pltpu.sync_copy(agg_sh.at[pl.ds(my_rows, RPT)],
                    aggp_hbm.at[c, pl.ds(my_rows, RPT)])


_agg_kernel = pl.kernel(
    _agg_body,
    out_type=jax.ShapeDtypeStruct((NC, NPAD, D), jnp.float32),
    mesh=_MESH,
    scratch_types=[
        pltpu.VMEM((ROWS_A, K), jnp.int32),
        pltpu.VMEM((NBUF, K), jnp.int32),
        pltpu.VMEM((NBUF, K), jnp.int32),
        pltpu.VMEM((NBUF, K, D), jnp.float32),
        pltpu.VMEM_SHARED((NPAD, D), jnp.float32),
    ] + [pltpu.SemaphoreType.DMA] * (2 * NBUF),
)


_BB = 512

def _xw_body(x_ref, w_ref, xw_ref):
    xw_ref[...] = jnp.dot(x_ref[...], w_ref[...],
                          preferred_element_type=jnp.float32)


_xw_call = pl.pallas_call(
    _xw_body,
    grid=(NPAD // _BB,),
    in_specs=[
        pl.BlockSpec((_BB, D), lambda i: (i, 0)),
        pl.BlockSpec((D, D), lambda i: (0, 0)),
    ],
    out_specs=pl.BlockSpec((_BB, D), lambda i: (i, 0)),
    out_shape=jax.ShapeDtypeStruct((NPAD, D), jnp.float32),
)


def _y_body(xw_ref, degp_ref, y_ref):
    deg = degp_ref[0, :] + degp_ref[1, :] + 1.0
    dinv = lax.rsqrt(deg)
    y_ref[...] = xw_ref[...] * dinv[:, None]


_y_call = pl.pallas_call(
    _y_body,
    grid=(NPAD // _BB,),
    in_specs=[
        pl.BlockSpec((_BB, D), lambda i: (i, 0)),
        pl.BlockSpec((NC, _BB), lambda i: (0, i)),
    ],
    out_specs=pl.BlockSpec((_BB, D), lambda i: (i, 0)),
    out_shape=jax.ShapeDtypeStruct((NPAD, D), jnp.float32),
)


_OB = 512

def _out_body(x_ref, aggp_ref, y_ref, degp_ref, b_ref, o_ref):
    ssum = aggp_ref[0] + aggp_ref[1] + y_ref[...]
    deg = degp_ref[0, :] + degp_ref[1, :] + 1.0
    dinv = lax.rsqrt(deg)
    conv = ssum * dinv[:, None] + b_ref[...]
    o_ref[...] = x_ref[...] + jnp.maximum(conv, 0.0)


_out_call = pl.pallas_call(
    _out_body,
    grid=(pl.cdiv(N, _OB),),
    in_specs=[
        pl.BlockSpec((_OB, D), lambda i: (i, 0)),
        pl.BlockSpec((NC, _OB, D), lambda i: (0, i, 0)),
        pl.BlockSpec((_OB, D), lambda i: (i, 0)),
        pl.BlockSpec((NC, _OB), lambda i: (0, i)),
        pl.BlockSpec((1, D), lambda i: (0, 0)),
    ],
    out_specs=pl.BlockSpec((_OB, D), lambda i: (i, 0)),
    out_shape=jax.ShapeDtypeStruct((N, D), jnp.float32),
)


def kernel(x, edge_index, W, b):
    src = edge_index[0]
    dst = edge_index[1]
    pad = (N + (jnp.arange(EPAD - E, dtype=jnp.int32) % (NPAD - N))).astype(
        jnp.int32)
    src_p = jnp.concatenate([src, pad])
    dst_p = jnp.concatenate([dst, pad])
    packed = (src_p | (dst_p << 16)).reshape(EPAD // K, K)
    dst2 = dst_p.reshape(EPAD // K, K)

    degp = _deg_kernel(dst2)
    xw = _xw_call(x, W)
    y = _y_call(xw, degp)
    aggp = _agg_kernel(packed, y)
    return _out_call(x, aggp, y, degp, b.reshape(1, D))


# speedup vs baseline: 1.0520x; 1.0520x over previous
import jax
import jax.numpy as jnp
from jax import lax
from jax.experimental import pallas as pl
from jax.experimental.pallas import tpu as pltpu
from jax.experimental.pallas import tpu_sc as plsc

N = 10000
D = 128
E = 320000
DH = D // 2

NC = 2
NS = 16
NW = NC * NS
NPAD = 10240
EPAD = 327680
K = 128
ROWS_A = EPAD // K // NW
ROWS_C = EPAD // K // NS
RPT = NPAD // NS

_MESH = plsc.VectorSubcoreMesh(
    core_axis_name="c", subcore_axis_name="s", num_cores=NC, num_subcores=NS
)


def _deg_body(dst2_hbm, degp_hbm, didx_v, ones_v, zrow_v, hist_sh, asem):
    c = lax.axis_index("c")
    s = lax.axis_index("s")
    _ZV = jnp.zeros((16,), jnp.float32)
    _OV = jnp.ones((16,), jnp.float32)

    def fill_o(i, carry):
        ones_v[pl.ds(i * 16, 16)] = _OV
        return carry

    lax.fori_loop(0, K // 16, fill_o, 0)

    def fill_z(i, carry):
        zrow_v[pl.ds(i * 16, 16)] = _ZV
        return carry

    lax.fori_loop(0, RPT // 16, fill_z, 0)

    my_rows = pl.multiple_of(s * RPT, 8)
    pltpu.sync_copy(zrow_v, hist_sh.at[pl.ds(my_rows, RPT)])
    row0 = pl.multiple_of((c * NS + s) * ROWS_A, 8)
    pltpu.sync_copy(dst2_hbm.at[pl.ds(row0, ROWS_A)], didx_v)
    plsc.subcore_barrier()

    def fire(k, carry):
        pltpu.async_copy(ones_v, hist_sh.at[didx_v.at[k]], asem, add=True)
        return carry

    lax.fori_loop(0, ROWS_A, fire, 0)

    def drain(k, carry):
        pltpu.make_async_copy(ones_v, hist_sh.at[didx_v.at[0]], asem).wait()
        return carry

    lax.fori_loop(0, ROWS_A, drain, 0)
    plsc.subcore_barrier()
    pltpu.sync_copy(hist_sh.at[pl.ds(my_rows, RPT)],
                    degp_hbm.at[c, pl.ds(my_rows, RPT)])


_deg_kernel = pl.kernel(
    _deg_body,
    out_type=jax.ShapeDtypeStruct((NC, NPAD), jnp.float32),
    mesh=_MESH,
    scratch_types=[
        pltpu.VMEM((ROWS_A, K), jnp.int32),
        pltpu.VMEM((K,), jnp.float32),
        pltpu.VMEM((RPT,), jnp.float32),
        pltpu.VMEM_SHARED((NPAD,), jnp.float32),
        pltpu.SemaphoreType.DMA,
    ],
)


NBUF = 2
G = ROWS_A // NBUF

def _agg_body(packed_hbm, y_hbm, aggp_hbm, packed_v, sidx_v, didx_v, rows_v,
              agg_sh, *sems):
    gsem = sems[:NBUF]
    ssem = sems[NBUF:]
    c = lax.axis_index("c")
    s = lax.axis_index("s")
    _ZV = jnp.zeros((16,), jnp.float32)

    def fz(i, carry):
        def fz2(j, carry2):
            rows_v[0, i, pl.ds(j * 16, 16)] = _ZV
            return carry2
        return lax.fori_loop(0, D // 16, fz2, carry)

    lax.fori_loop(0, K, fz, 0)

    my_rows = pl.multiple_of(s * RPT, 8)

    def fzd(j, carry):
        off = pl.multiple_of(my_rows + j * K, 8)
        pltpu.sync_copy(rows_v.at[0], agg_sh.at[pl.ds(off, K)])
        return carry

    lax.fori_loop(0, RPT // K, fzd, 0)

    row0 = pl.multiple_of((c * NS + s) * ROWS_A, 8)
    pltpu.sync_copy(packed_hbm.at[pl.ds(row0, ROWS_A)], packed_v)
    plsc.subcore_barrier()

    def unpack(k, b):
        def up(j, carry):
            p = packed_v[k, pl.ds(j * 16, 16)]
            sidx_v[b, pl.ds(j * 16, 16)] = p & 0xFFFF
            didx_v[b, pl.ds(j * 16, 16)] = lax.shift_right_logical(p, 16)
            return carry
        lax.fori_loop(0, K // 16, up, 0)

    for b in range(NBUF):
        unpack(b, b)
        pltpu.async_copy(y_hbm.at[sidx_v.at[b]], rows_v.at[b], gsem[b])

    def outer(g, carry):
        for b in range(NBUF):
            k = g * NBUF + b
            pltpu.make_async_copy(y_hbm.at[sidx_v.at[b]], rows_v.at[b],
                                  gsem[b]).wait()
            pltpu.async_copy(rows_v.at[b], agg_sh.at[didx_v.at[b]], ssem[b],
                             add=True)
            pltpu.make_async_copy(rows_v.at[b], agg_sh.at[didx_v.at[b]],
                                  ssem[b]).wait()
            unpack(k + NBUF, b)
            pltpu.async_copy(y_hbm.at[sidx_v.at[b]], rows_v.at[b], gsem[b])
        return carry

    lax.fori_loop(0, G - 1, outer, 0)
    for b in range(NBUF):
        pltpu.make_async_copy(y_hbm.at[sidx_v.at[b]], rows_v.at[b],
                              gsem[b]).wait()
        pltpu.async_copy(rows_v.at[b], agg_sh.at[didx_v.at[b]], ssem[b],
                         add=True)
    for b in range(NBUF):
        pltpu.make_async_copy(rows_v.at[b], agg_sh.at[didx_v.at[b]],
                              ssem[b]).wait()

    plsc.subcore_barrier()
    pltpu.sync_copy(agg_sh.at[pl.ds(my_rows, RPT)],
                    aggp_hbm.at[c, pl.ds(my_rows, RPT)])


_agg_kernel = pl.kernel(
    _agg_body,
    out_type=jax.ShapeDtypeStruct((NC, NPAD, D), jnp.float32),
    mesh=_MESH,
    scratch_types=[
        pltpu.VMEM((ROWS_A, K), jnp.int32),
        pltpu.VMEM((NBUF, K), jnp.int32),
        pltpu.VMEM((NBUF, K), jnp.int32),
        pltpu.VMEM((NBUF, K, D), jnp.float32),
        pltpu.VMEM_SHARED((NPAD, D), jnp.float32),
    ] + [pltpu.SemaphoreType.DMA] * (2 * NBUF),
)


_BB = 512

def _y_body(x_ref, w_ref, degp_ref, y_ref):
    xw = jnp.dot(x_ref[...], w_ref[...], preferred_element_type=jnp.float32)
    deg = degp_ref[0, :] + degp_ref[1, :] + 1.0
    dinv = lax.rsqrt(deg)
    y_ref[...] = xw * dinv[:, None]


_y_call = pl.pallas_call(
    _y_body,
    grid=(NPAD // _BB,),
    in_specs=[
        pl.BlockSpec((_BB, D), lambda i: (i, 0)),
        pl.BlockSpec((D, D), lambda i: (0, 0)),
        pl.BlockSpec((NC, _BB), lambda i: (0, i)),
    ],
    out_specs=pl.BlockSpec((_BB, D), lambda i: (i, 0)),
    out_shape=jax.ShapeDtypeStruct((NPAD, D), jnp.float32),
)


_OB = 512

def _out_body(x_ref, aggp_ref, y_ref, degp_ref, b_ref, o_ref):
    ssum = aggp_ref[0] + aggp_ref[1] + y_ref[...]
    deg = degp_ref[0, :] + degp_ref[1, :] + 1.0
    dinv = lax.rsqrt(deg)
    conv = ssum * dinv[:, None] + b_ref[...]
    o_ref[...] = x_ref[...] + jnp.maximum(conv, 0.0)


_out_call = pl.pallas_call(
    _out_body,
    grid=(pl.cdiv(N, _OB),),
    in_specs=[
        pl.BlockSpec((_OB, D), lambda i: (i, 0)),
        pl.BlockSpec((NC, _OB, D), lambda i: (0, i, 0)),
        pl.BlockSpec((_OB, D), lambda i: (i, 0)),
        pl.BlockSpec((NC, _OB), lambda i: (0, i)),
        pl.BlockSpec((1, D), lambda i: (0, 0)),
    ],
    out_specs=pl.BlockSpec((_OB, D), lambda i: (i, 0)),
    out_shape=jax.ShapeDtypeStruct((N, D), jnp.float32),
)


def kernel(x, edge_index, W, b):
    src = edge_index[0]
    dst = edge_index[1]
    pad = (N + (jnp.arange(EPAD - E, dtype=jnp.int32) % (NPAD - N))).astype(
        jnp.int32)
    src_p = jnp.concatenate([src, pad])
    dst_p = jnp.concatenate([dst, pad])
    packed = (src_p | (dst_p << 16)).reshape(EPAD // K, K)
    dst2 = dst_p.reshape(EPAD // K, K)

    degp = _deg_kernel(dst2)
    y = _y_call(x, W, degp)
    aggp = _agg_kernel(packed, y)
    return _out_call(x, aggp, y, degp, b.reshape(1, D))


# speedup vs baseline: 1.1085x; 1.0537x over previous
import jax
import jax.numpy as jnp
import numpy as np
from jax import lax
from jax.experimental import pallas as pl
from jax.experimental.pallas import tpu as pltpu
from jax.experimental.pallas import tpu_sc as plsc

N = 10000
D = 128
E = 320000
DH = D // 2

NC = 2
NS = 16
NW = NC * NS
NPAD = 10240
EPAD = 327680
K = 128
ROWS_A = EPAD // K // NW
ROWS_C = EPAD // K // NS
RPT = NPAD // NS

_MESH = plsc.VectorSubcoreMesh(
    core_axis_name="c", subcore_axis_name="s", num_cores=NC, num_subcores=NS
)


def _deg_body(packed_hbm, degp_hbm, pk_v, didx_v, ones_v, zrow_v, hist_sh,
              asem):
    c = lax.axis_index("c")
    s = lax.axis_index("s")
    _ZV = jnp.zeros((16,), jnp.float32)
    _OV = jnp.ones((16,), jnp.float32)

    row0 = pl.multiple_of((c * NS + s) * ROWS_A, 8)
    pltpu.async_copy(packed_hbm.at[pl.ds(row0, ROWS_A)], pk_v, asem)

    def fill_o(i, carry):
        ones_v[pl.ds(i * 16, 16)] = _OV
        return carry

    lax.fori_loop(0, K // 16, fill_o, 0)

    def fill_z(i, carry):
        zrow_v[pl.ds(i * 16, 16)] = _ZV
        return carry

    lax.fori_loop(0, RPT // 16, fill_z, 0)

    my_rows = pl.multiple_of(s * RPT, 8)
    pltpu.sync_copy(zrow_v, hist_sh.at[pl.ds(my_rows, RPT)])
    pltpu.make_async_copy(packed_hbm.at[pl.ds(row0, ROWS_A)], pk_v,
                          asem).wait()

    def unp(i, carry):
        def up2(j, carry2):
            p = pk_v[i, pl.ds(j * 16, 16)]
            didx_v[i, pl.ds(j * 16, 16)] = lax.shift_right_logical(p, 16)
            return carry2
        return lax.fori_loop(0, K // 16, up2, carry)

    lax.fori_loop(0, ROWS_A, unp, 0)
    plsc.subcore_barrier()

    def fire(k, carry):
        pltpu.async_copy(ones_v, hist_sh.at[didx_v.at[k]], asem, add=True)
        return carry

    lax.fori_loop(0, ROWS_A, fire, 0)

    def drain(k, carry):
        pltpu.make_async_copy(ones_v, hist_sh.at[didx_v.at[0]], asem).wait()
        return carry

    lax.fori_loop(0, ROWS_A, drain, 0)
    plsc.subcore_barrier()
    pltpu.sync_copy(hist_sh.at[pl.ds(my_rows, RPT)],
                    degp_hbm.at[c, pl.ds(my_rows, RPT)])


_deg_kernel = pl.kernel(
    _deg_body,
    out_type=jax.ShapeDtypeStruct((NC, NPAD), jnp.float32),
    mesh=_MESH,
    scratch_types=[
        pltpu.VMEM((ROWS_A, K), jnp.int32),
        pltpu.VMEM((ROWS_A, K), jnp.int32),
        pltpu.VMEM((K,), jnp.float32),
        pltpu.VMEM((RPT,), jnp.float32),
        pltpu.VMEM_SHARED((NPAD,), jnp.float32),
        pltpu.SemaphoreType.DMA,
    ],
)


NBUF = 2
G = ROWS_A // NBUF

def _agg_body(packed_hbm, y_hbm, aggp_hbm, packed_v, sidx_v, didx_v, rows_v,
              agg_sh, *sems):
    gsem = sems[:NBUF]
    ssem = sems[NBUF:]
    c = lax.axis_index("c")
    s = lax.axis_index("s")
    _ZV = jnp.zeros((16,), jnp.float32)

    def fz(i, carry):
        def fz2(j, carry2):
            rows_v[0, i, pl.ds(j * 16, 16)] = _ZV
            return carry2
        return lax.fori_loop(0, D // 16, fz2, carry)

    lax.fori_loop(0, K, fz, 0)

    my_rows = pl.multiple_of(s * RPT, 8)

    def fzd(j, carry):
        off = pl.multiple_of(my_rows + j * K, 8)
        pltpu.sync_copy(rows_v.at[0], agg_sh.at[pl.ds(off, K)])
        return carry

    lax.fori_loop(0, RPT // K, fzd, 0)

    row0 = pl.multiple_of((c * NS + s) * ROWS_A, 8)
    pltpu.sync_copy(packed_hbm.at[pl.ds(row0, ROWS_A)], packed_v)
    plsc.subcore_barrier()

    def unpack(k, b):
        def up(j, carry):
            p = packed_v[k, pl.ds(j * 16, 16)]
            sidx_v[b, pl.ds(j * 16, 16)] = p & 0xFFFF
            didx_v[b, pl.ds(j * 16, 16)] = lax.shift_right_logical(p, 16)
            return carry
        lax.fori_loop(0, K // 16, up, 0)

    for b in range(NBUF):
        unpack(b, b)
        pltpu.async_copy(y_hbm.at[sidx_v.at[b]], rows_v.at[b], gsem[b])

    def outer(g, carry):
        for b in range(NBUF):
            k = g * NBUF + b
            pltpu.make_async_copy(y_hbm.at[sidx_v.at[b]], rows_v.at[b],
                                  gsem[b]).wait()
            pltpu.async_copy(rows_v.at[b], agg_sh.at[didx_v.at[b]], ssem[b],
                             add=True)
            pltpu.make_async_copy(rows_v.at[b], agg_sh.at[didx_v.at[b]],
                                  ssem[b]).wait()
            unpack(k + NBUF, b)
            pltpu.async_copy(y_hbm.at[sidx_v.at[b]], rows_v.at[b], gsem[b])
        return carry

    lax.fori_loop(0, G - 1, outer, 0)
    for b in range(NBUF):
        pltpu.make_async_copy(y_hbm.at[sidx_v.at[b]], rows_v.at[b],
                              gsem[b]).wait()
        pltpu.async_copy(rows_v.at[b], agg_sh.at[didx_v.at[b]], ssem[b],
                         add=True)
    for b in range(NBUF):
        pltpu.make_async_copy(rows_v.at[b], agg_sh.at[didx_v.at[b]],
                              ssem[b]).wait()

    plsc.subcore_barrier()
    pltpu.sync_copy(agg_sh.at[pl.ds(my_rows, RPT)],
                    aggp_hbm.at[c, pl.ds(my_rows, RPT)])


_agg_kernel = pl.kernel(
    _agg_body,
    out_type=jax.ShapeDtypeStruct((NC, NPAD, D), jnp.float32),
    mesh=_MESH,
    scratch_types=[
        pltpu.VMEM((ROWS_A, K), jnp.int32),
        pltpu.VMEM((NBUF, K), jnp.int32),
        pltpu.VMEM((NBUF, K), jnp.int32),
        pltpu.VMEM((NBUF, K, D), jnp.float32),
        pltpu.VMEM_SHARED((NPAD, D), jnp.float32),
    ] + [pltpu.SemaphoreType.DMA] * (2 * NBUF),
)


_BB = 1024

def _y_body(x_ref, w_ref, degp_ref, y_ref):
    xw = jnp.dot(x_ref[...], w_ref[...], preferred_element_type=jnp.float32)
    deg = degp_ref[0, :] + degp_ref[1, :] + 1.0
    dinv = lax.rsqrt(deg)
    y_ref[...] = xw * dinv[:, None]


_y_call = pl.pallas_call(
    _y_body,
    grid=(NPAD // _BB,),
    in_specs=[
        pl.BlockSpec((_BB, D), lambda i: (i, 0)),
        pl.BlockSpec((D, D), lambda i: (0, 0)),
        pl.BlockSpec((NC, _BB), lambda i: (0, i)),
    ],
    out_specs=pl.BlockSpec((_BB, D), lambda i: (i, 0)),
    out_shape=jax.ShapeDtypeStruct((NPAD, D), jnp.float32),
)


_OB = 1024

def _out_body(x_ref, aggp_ref, y_ref, degp_ref, b_ref, o_ref):
    ssum = aggp_ref[0] + aggp_ref[1] + y_ref[...]
    deg = degp_ref[0, :] + degp_ref[1, :] + 1.0
    dinv = lax.rsqrt(deg)
    conv = ssum * dinv[:, None] + b_ref[...]
    o_ref[...] = x_ref[...] + jnp.maximum(conv, 0.0)


_out_call = pl.pallas_call(
    _out_body,
    grid=(pl.cdiv(N, _OB),),
    in_specs=[
        pl.BlockSpec((_OB, D), lambda i: (i, 0)),
        pl.BlockSpec((NC, _OB, D), lambda i: (0, i, 0)),
        pl.BlockSpec((_OB, D), lambda i: (i, 0)),
        pl.BlockSpec((NC, _OB), lambda i: (0, i)),
        pl.BlockSpec((1, D), lambda i: (0, 0)),
    ],
    out_specs=pl.BlockSpec((_OB, D), lambda i: (i, 0)),
    out_shape=jax.ShapeDtypeStruct((N, D), jnp.float32),
)


_PAD = N + (np.arange(EPAD - E, dtype=np.int64) % (NPAD - N))
_PAD_PACKED = (_PAD | (_PAD << 16)).astype(np.int32)


def kernel(x, edge_index, W, b):
    src = edge_index[0]
    dst = edge_index[1]
    packed = jnp.concatenate(
        [src | (dst << 16), jnp.asarray(_PAD_PACKED)]).reshape(EPAD // K, K)

    degp = _deg_kernel(packed)
    y = _y_call(x, W, degp)
    aggp = _agg_kernel(packed, y)
    return _out_call(x, aggp, y, degp, b.reshape(1, D))


# speedup vs baseline: 1.1914x; 1.0748x over previous
import jax
import jax.numpy as jnp
import numpy as np
from jax import lax
from jax.experimental import pallas as pl
from jax.experimental.pallas import tpu as pltpu
from jax.experimental.pallas import tpu_sc as plsc

N = 10000
D = 128
E = 320000
DH = D // 2

NC = 2
NS = 16
NW = NC * NS
NPAD = 10240
EPAD = 327680
K = 128
ROWS_A = EPAD // K // NW
ROWS_C = EPAD // K // NS
RPT = NPAD // NS

_MESH = plsc.VectorSubcoreMesh(
    core_axis_name="c", subcore_axis_name="s", num_cores=NC, num_subcores=NS
)


def _deg_body(ei3_hbm, degp_hbm, packed_hbm, sidx_v, didx_v, ones_v, zrow_v,
              hist_sh, asem, psem):
    c = lax.axis_index("c")
    s = lax.axis_index("s")
    w = c * NS + s
    _ZV = jnp.zeros((16,), jnp.float32)
    _OV = jnp.ones((16,), jnp.float32)

    row0 = pl.multiple_of(w * ROWS_A, 8)
    pltpu.async_copy(ei3_hbm.at[0, pl.ds(row0, ROWS_A)], sidx_v, asem)
    pltpu.async_copy(ei3_hbm.at[1, pl.ds(row0, ROWS_A)], didx_v, asem)

    def fill_o(i, carry):
        ones_v[pl.ds(i * 16, 16)] = _OV
        return carry

    lax.fori_loop(0, K // 16, fill_o, 0)

    def fill_z(i, carry):
        zrow_v[pl.ds(i * 16, 16)] = _ZV
        return carry

    lax.fori_loop(0, RPT // 16, fill_z, 0)

    my_rows = pl.multiple_of(s * RPT, 8)
    pltpu.sync_copy(zrow_v, hist_sh.at[pl.ds(my_rows, RPT)])

    pltpu.make_async_copy(ei3_hbm.at[0, pl.ds(row0, ROWS_A)], sidx_v,
                          asem).wait()
    pltpu.make_async_copy(ei3_hbm.at[1, pl.ds(row0, ROWS_A)], didx_v,
                          asem).wait()
    plsc.subcore_barrier()

    def fire(k, carry):
        pltpu.async_copy(ones_v, hist_sh.at[didx_v.at[k]], asem, add=True)
        return carry

    lax.fori_loop(0, ROWS_A, fire, 0)

    def pack_i(i, carry):
        def pack_j(j, carry2):
            sl = pl.ds(j * 16, 16)
            sidx_v[i, sl] = sidx_v[i, sl] | (didx_v[i, sl] << 16)
            return carry2
        return lax.fori_loop(0, K // 16, pack_j, carry)

    lax.fori_loop(0, ROWS_A, pack_i, 0)
    pltpu.async_copy(sidx_v, packed_hbm.at[pl.ds(row0, ROWS_A)], psem)

    def drain(k, carry):
        pltpu.make_async_copy(ones_v, hist_sh.at[didx_v.at[0]], asem).wait()
        return carry

    lax.fori_loop(0, ROWS_A, drain, 0)
    plsc.subcore_barrier()
    pltpu.sync_copy(hist_sh.at[pl.ds(my_rows, RPT)],
                    degp_hbm.at[c, pl.ds(my_rows, RPT)])
    pltpu.make_async_copy(sidx_v, packed_hbm.at[pl.ds(row0, ROWS_A)],
                          psem).wait()


_deg_kernel = pl.kernel(
    _deg_body,
    out_type=(
        jax.ShapeDtypeStruct((NC, NPAD), jnp.float32),
        jax.ShapeDtypeStruct((EPAD // K, K), jnp.int32),
    ),
    mesh=_MESH,
    scratch_types=[
        pltpu.VMEM((ROWS_A, K), jnp.int32),
        pltpu.VMEM((ROWS_A, K), jnp.int32),
        pltpu.VMEM((K,), jnp.float32),
        pltpu.VMEM((RPT,), jnp.float32),
        pltpu.VMEM_SHARED((NPAD,), jnp.float32),
        pltpu.SemaphoreType.DMA,
        pltpu.SemaphoreType.DMA,
    ],
)


NBUF = 2
G = ROWS_A // NBUF

def _agg_body(packed_hbm, y_hbm, aggp_hbm, packed_v, sidx_v, didx_v, rows_v,
              agg_sh, *sems):
    gsem = sems[:NBUF]
    ssem = sems[NBUF:]
    c = lax.axis_index("c")
    s = lax.axis_index("s")
    _ZV = jnp.zeros((16,), jnp.float32)

    row0 = pl.multiple_of((c * NS + s) * ROWS_A, 8)
    pltpu.async_copy(packed_hbm.at[pl.ds(row0, ROWS_A)], packed_v, gsem[0])

    def fz(i, carry):
        def fz2(j, carry2):
            rows_v[0, i, pl.ds(j * 16, 16)] = _ZV
            return carry2
        return lax.fori_loop(0, D // 16, fz2, carry)

    lax.fori_loop(0, K, fz, 0)

    my_rows = pl.multiple_of(s * RPT, 8)

    def fzd(j, carry):
        off = pl.multiple_of(my_rows + j * K, 8)
        pltpu.async_copy(rows_v.at[0], agg_sh.at[pl.ds(off, K)], ssem[0])
        return carry

    lax.fori_loop(0, RPT // K, fzd, 0)

    pltpu.make_async_copy(packed_hbm.at[pl.ds(row0, ROWS_A)], packed_v,
                          gsem[0]).wait()

    def fzw(j, carry):
        off = pl.multiple_of(my_rows + j * K, 8)
        pltpu.make_async_copy(rows_v.at[0], agg_sh.at[pl.ds(off, K)],
                              ssem[0]).wait()
        return carry

    lax.fori_loop(0, RPT // K, fzw, 0)
    plsc.subcore_barrier()

    def unpack(k, b):
        def up(j, carry):
            p = packed_v[k, pl.ds(j * 16, 16)]
            sidx_v[b, pl.ds(j * 16, 16)] = p & 0xFFFF
            didx_v[b, pl.ds(j * 16, 16)] = lax.shift_right_logical(p, 16)
            return carry
        lax.fori_loop(0, K // 16, up, 0)

    for b in range(NBUF):
        unpack(b, b)
        pltpu.async_copy(y_hbm.at[sidx_v.at[b]], rows_v.at[b], gsem[b])

    def outer(g, carry):
        for b in range(NBUF):
            k = g * NBUF + b
            pltpu.make_async_copy(y_hbm.at[sidx_v.at[b]], rows_v.at[b],
                                  gsem[b]).wait()
            pltpu.async_copy(rows_v.at[b], agg_sh.at[didx_v.at[b]], ssem[b],
                             add=True)
            pltpu.make_async_copy(rows_v.at[b], agg_sh.at[didx_v.at[b]],
                                  ssem[b]).wait()
            unpack(k + NBUF, b)
            pltpu.async_copy(y_hbm.at[sidx_v.at[b]], rows_v.at[b], gsem[b])
        return carry

    lax.fori_loop(0, G - 1, outer, 0)
    for b in range(NBUF):
        pltpu.make_async_copy(y_hbm.at[sidx_v.at[b]], rows_v.at[b],
                              gsem[b]).wait()
        pltpu.async_copy(rows_v.at[b], agg_sh.at[didx_v.at[b]], ssem[b],
                         add=True)
    for b in range(NBUF):
        pltpu.make_async_copy(rows_v.at[b], agg_sh.at[didx_v.at[b]],
                              ssem[b]).wait()

    plsc.subcore_barrier()
    pltpu.sync_copy(agg_sh.at[pl.ds(my_rows, RPT)],
                    aggp_hbm.at[c, pl.ds(my_rows, RPT)])


_agg_kernel = pl.kernel(
    _agg_body,
    out_type=jax.ShapeDtypeStruct((NC, NPAD, D), jnp.float32),
    mesh=_MESH,
    scratch_types=[
        pltpu.VMEM((ROWS_A, K), jnp.int32),
        pltpu.VMEM((NBUF, K), jnp.int32),
        pltpu.VMEM((NBUF, K), jnp.int32),
        pltpu.VMEM((NBUF, K, D), jnp.float32),
        pltpu.VMEM_SHARED((NPAD, D), jnp.float32),
    ] + [pltpu.SemaphoreType.DMA] * (2 * NBUF),
)


_BB = 1024

def _y_body(x_ref, w_ref, degp_ref, y_ref):
    xw = jnp.dot(x_ref[...], w_ref[...], preferred_element_type=jnp.float32)
    deg = degp_ref[0, :] + degp_ref[1, :] + 1.0
    dinv = lax.rsqrt(deg)
    y_ref[...] = xw * dinv[:, None]


_y_call = pl.pallas_call(
    _y_body,
    grid=(NPAD // _BB,),
    in_specs=[
        pl.BlockSpec((_BB, D), lambda i: (i, 0)),
        pl.BlockSpec((D, D), lambda i: (0, 0)),
        pl.BlockSpec((NC, _BB), lambda i: (0, i)),
    ],
    out_specs=pl.BlockSpec((_BB, D), lambda i: (i, 0)),
    out_shape=jax.ShapeDtypeStruct((NPAD, D), jnp.float32),
)


_OB = 1024

def _out_body(x_ref, aggp_ref, y_ref, degp_ref, b_ref, o_ref):
    ssum = aggp_ref[0] + aggp_ref[1] + y_ref[...]
    deg = degp_ref[0, :] + degp_ref[1, :] + 1.0
    dinv = lax.rsqrt(deg)
    conv = ssum * dinv[:, None] + b_ref[...]
    o_ref[...] = x_ref[...] + jnp.maximum(conv, 0.0)


_out_call = pl.pallas_call(
    _out_body,
    grid=(pl.cdiv(N, _OB),),
    in_specs=[
        pl.BlockSpec((_OB, D), lambda i: (i, 0)),
        pl.BlockSpec((NC, _OB, D), lambda i: (0, i, 0)),
        pl.BlockSpec((_OB, D), lambda i: (i, 0)),
        pl.BlockSpec((NC, _OB), lambda i: (0, i)),
        pl.BlockSpec((1, D), lambda i: (0, 0)),
    ],
    out_specs=pl.BlockSpec((_OB, D), lambda i: (i, 0)),
    out_shape=jax.ShapeDtypeStruct((N, D), jnp.float32),
)


_PAD = np.broadcast_to(
    (N + (np.arange(EPAD - E, dtype=np.int64) % (NPAD - N))).astype(
        np.int32).reshape(1, (EPAD - E) // K, K),
    (2, (EPAD - E) // K, K)).copy()


def kernel(x, edge_index, W, b):
    ei3 = jnp.concatenate(
        [edge_index.reshape(2, E // K, K), jnp.asarray(_PAD)], axis=1)
    degp, packed = _deg_kernel(ei3)
    y = _y_call(x, W, degp)
    aggp = _agg_kernel(packed, y)
    return _out_call(x, aggp, y, degp, b.reshape(1, D))


# speedup vs baseline: 1.2776x; 1.0723x over previous
import jax
import jax.numpy as jnp
import numpy as np
from jax import lax
from jax.experimental import pallas as pl
from jax.experimental.pallas import tpu as pltpu
from jax.experimental.pallas import tpu_sc as plsc

N = 10000
D = 128
E = 320000
DH = D // 2

NC = 2
NS = 16
NW = NC * NS
NPAD = 10240
EPAD = 327680
K = 128
ROWS_A = EPAD // K // NW
ROWS_C = EPAD // K // NS
RPT = NPAD // NS

_MESH = plsc.VectorSubcoreMesh(
    core_axis_name="c", subcore_axis_name="s", num_cores=NC, num_subcores=NS
)


def _deg_body(ei3_hbm, degp_hbm, packed_hbm, sidx_v, didx_v, ones_v, zrow_v,
              hist_sh, asem, psem):
    c = lax.axis_index("c")
    s = lax.axis_index("s")
    w = c * NS + s
    _ZV = jnp.zeros((16,), jnp.float32)
    _OV = jnp.ones((16,), jnp.float32)

    row0 = pl.multiple_of(w * ROWS_A, 8)
    pltpu.async_copy(ei3_hbm.at[0, pl.ds(row0, ROWS_A)], sidx_v, asem)
    pltpu.async_copy(ei3_hbm.at[1, pl.ds(row0, ROWS_A)], didx_v, asem)

    def fill_o(i, carry):
        ones_v[pl.ds(i * 16, 16)] = _OV
        return carry

    lax.fori_loop(0, K // 16, fill_o, 0)

    def fill_z(i, carry):
        zrow_v[pl.ds(i * 16, 16)] = _ZV
        return carry

    lax.fori_loop(0, RPT // 16, fill_z, 0)

    my_rows = pl.multiple_of(s * RPT, 8)
    pltpu.sync_copy(zrow_v, hist_sh.at[pl.ds(my_rows, RPT)])

    pltpu.make_async_copy(ei3_hbm.at[0, pl.ds(row0, ROWS_A)], sidx_v,
                          asem).wait()
    pltpu.make_async_copy(ei3_hbm.at[1, pl.ds(row0, ROWS_A)], didx_v,
                          asem).wait()
    plsc.subcore_barrier()

    def fire(k, carry):
        pltpu.async_copy(ones_v, hist_sh.at[didx_v.at[k]], asem, add=True)
        return carry

    lax.fori_loop(0, ROWS_A, fire, 0)

    def pack_i(i, carry):
        def pack_j(j, carry2):
            sl = pl.ds(j * 16, 16)
            sidx_v[i, sl] = sidx_v[i, sl] | (didx_v[i, sl] << 16)
            return carry2
        return lax.fori_loop(0, K // 16, pack_j, carry)

    lax.fori_loop(0, ROWS_A, pack_i, 0)
    pltpu.async_copy(sidx_v, packed_hbm.at[pl.ds(row0, ROWS_A)], psem)

    def drain(k, carry):
        pltpu.make_async_copy(ones_v, hist_sh.at[didx_v.at[0]], asem).wait()
        return carry

    lax.fori_loop(0, ROWS_A, drain, 0)
    plsc.subcore_barrier()
    pltpu.sync_copy(hist_sh.at[pl.ds(my_rows, RPT)],
                    degp_hbm.at[c, pl.ds(my_rows, RPT)])
    pltpu.make_async_copy(sidx_v, packed_hbm.at[pl.ds(row0, ROWS_A)],
                          psem).wait()


_deg_kernel = pl.kernel(
    _deg_body,
    out_type=(
        jax.ShapeDtypeStruct((NC, NPAD), jnp.float32),
        jax.ShapeDtypeStruct((EPAD // K, K), jnp.int32),
    ),
    mesh=_MESH,
    scratch_types=[
        pltpu.VMEM((ROWS_A, K), jnp.int32),
        pltpu.VMEM((ROWS_A, K), jnp.int32),
        pltpu.VMEM((K,), jnp.float32),
        pltpu.VMEM((RPT,), jnp.float32),
        pltpu.VMEM_SHARED((NPAD,), jnp.float32),
        pltpu.SemaphoreType.DMA,
        pltpu.SemaphoreType.DMA,
    ],
)


KC = 64
CROWS = EPAD // KC // NW
NBUF = 4
G = CROWS // NBUF

def _agg_body(packed_hbm, y_hbm, aggp_hbm, packed_v, sidx_v, didx_v, rows_v,
              agg_sh, *sems):
    gsem = sems[:NBUF]
    ssem = sems[NBUF:]
    c = lax.axis_index("c")
    s = lax.axis_index("s")
    _ZV = jnp.zeros((16,), jnp.float32)

    row0 = pl.multiple_of((c * NS + s) * CROWS, 8)
    pltpu.async_copy(packed_hbm.at[pl.ds(row0, CROWS // 2)], packed_v,
                     gsem[0])

    def fz(i, carry):
        def fz2(j, carry2):
            rows_v[0, i, pl.ds(j * 16, 16)] = _ZV
            return carry2
        return lax.fori_loop(0, D // 16, fz2, carry)

    lax.fori_loop(0, KC, fz, 0)

    my_rows = pl.multiple_of(s * RPT, 8)

    def fzd(j, carry):
        off = pl.multiple_of(my_rows + j * KC, 8)
        pltpu.async_copy(rows_v.at[0], agg_sh.at[pl.ds(off, KC)], ssem[0])
        return carry

    lax.fori_loop(0, RPT // KC, fzd, 0)

    pltpu.make_async_copy(packed_hbm.at[pl.ds(row0, CROWS // 2)], packed_v,
                          gsem[0]).wait()

    def fzw(j, carry):
        off = pl.multiple_of(my_rows + j * KC, 8)
        pltpu.make_async_copy(rows_v.at[0], agg_sh.at[pl.ds(off, KC)],
                              ssem[0]).wait()
        return carry

    lax.fori_loop(0, RPT // KC, fzw, 0)
    plsc.subcore_barrier()

    def unpack(k, b):
        kb = lax.rem(k, CROWS // 2)

        def up(j, carry):
            p = packed_v[kb, pl.ds(j * 16, 16)]
            sidx_v[b, pl.ds(j * 16, 16)] = p & 0xFFFF
            didx_v[b, pl.ds(j * 16, 16)] = lax.shift_right_logical(p, 16)
            return carry
        lax.fori_loop(0, KC // 16, up, 0)

    for b in range(NBUF):
        unpack(b, b)
        pltpu.async_copy(y_hbm.at[sidx_v.at[b]], rows_v.at[b], gsem[b])

    def outer(g, carry):
        @pl.when(g == CROWS // 2 // NBUF - 1)
        def _():
            pltpu.sync_copy(
                packed_hbm.at[pl.ds(row0 + CROWS // 2, CROWS // 2)], packed_v)

        for b in range(NBUF):
            k = g * NBUF + b
            pltpu.make_async_copy(y_hbm.at[sidx_v.at[b]], rows_v.at[b],
                                  gsem[b]).wait()
            pltpu.async_copy(rows_v.at[b], agg_sh.at[didx_v.at[b]], ssem[b],
                             add=True)
            pltpu.make_async_copy(rows_v.at[b], agg_sh.at[didx_v.at[b]],
                                  ssem[b]).wait()
            unpack(k + NBUF, b)
            pltpu.async_copy(y_hbm.at[sidx_v.at[b]], rows_v.at[b], gsem[b])
        return carry

    lax.fori_loop(0, G - 1, outer, 0)
    for b in range(NBUF):
        pltpu.make_async_copy(y_hbm.at[sidx_v.at[b]], rows_v.at[b],
                              gsem[b]).wait()
        pltpu.async_copy(rows_v.at[b], agg_sh.at[didx_v.at[b]], ssem[b],
                         add=True)
    for b in range(NBUF):
        pltpu.make_async_copy(rows_v.at[b], agg_sh.at[didx_v.at[b]],
                              ssem[b]).wait()

    plsc.subcore_barrier()
    pltpu.sync_copy(agg_sh.at[pl.ds(my_rows, RPT)],
                    aggp_hbm.at[c, pl.ds(my_rows, RPT)])


_agg_kernel = pl.kernel(
    _agg_body,
    out_type=jax.ShapeDtypeStruct((NC, NPAD, D), jnp.float32),
    mesh=_MESH,
    scratch_types=[
        pltpu.VMEM((CROWS // 2, KC), jnp.int32),
        pltpu.VMEM((NBUF, KC), jnp.int32),
        pltpu.VMEM((NBUF, KC), jnp.int32),
        pltpu.VMEM((NBUF, KC, D), jnp.float32),
        pltpu.VMEM_SHARED((NPAD, D), jnp.float32),
    ] + [pltpu.SemaphoreType.DMA] * (2 * NBUF),
)


_BB = 1024

def _y_body(x_ref, w_ref, degp_ref, y_ref):
    xw = jnp.dot(x_ref[...], w_ref[...], preferred_element_type=jnp.float32)
    deg = degp_ref[0, :] + degp_ref[1, :] + 1.0
    dinv = lax.rsqrt(deg)
    y_ref[...] = xw * dinv[:, None]


_y_call = pl.pallas_call(
    _y_body,
    grid=(NPAD // _BB,),
    in_specs=[
        pl.BlockSpec((_BB, D), lambda i: (i, 0)),
        pl.BlockSpec((D, D), lambda i: (0, 0)),
        pl.BlockSpec((NC, _BB), lambda i: (0, i)),
    ],
    out_specs=pl.BlockSpec((_BB, D), lambda i: (i, 0)),
    out_shape=jax.ShapeDtypeStruct((NPAD, D), jnp.float32),
)


_OB = 1024

def _out_body(x_ref, aggp_ref, y_ref, degp_ref, b_ref, o_ref):
    ssum = aggp_ref[0] + aggp_ref[1] + y_ref[...]
    deg = degp_ref[0, :] + degp_ref[1, :] + 1.0
    dinv = lax.rsqrt(deg)
    conv = ssum * dinv[:, None] + b_ref[...]
    o_ref[...] = x_ref[...] + jnp.maximum(conv, 0.0)


_out_call = pl.pallas_call(
    _out_body,
    grid=(pl.cdiv(N, _OB),),
    in_specs=[
        pl.BlockSpec((_OB, D), lambda i: (i, 0)),
        pl.BlockSpec((NC, _OB, D), lambda i: (0, i, 0)),
        pl.BlockSpec((_OB, D), lambda i: (i, 0)),
        pl.BlockSpec((NC, _OB), lambda i: (0, i)),
        pl.BlockSpec((1, D), lambda i: (0, 0)),
    ],
    out_specs=pl.BlockSpec((_OB, D), lambda i: (i, 0)),
    out_shape=jax.ShapeDtypeStruct((N, D), jnp.float32),
)


_PAD = np.broadcast_to(
    (N + (np.arange(EPAD - E, dtype=np.int64) % (NPAD - N))).astype(
        np.int32).reshape(1, (EPAD - E) // K, K),
    (2, (EPAD - E) // K, K)).copy()


def kernel(x, edge_index, W, b):
    ei3 = jnp.concatenate(
        [edge_index.reshape(2, E // K, K), jnp.asarray(_PAD)], axis=1)
    degp, packed = _deg_kernel(ei3)
    y = _y_call(x, W, degp)
    aggp = _agg_kernel(packed.reshape(EPAD // KC, KC), y)
    return _out_call(x, aggp, y, degp, b.reshape(1, D))


# speedup vs baseline: 1.3049x; 1.0214x over previous
import jax
import jax.numpy as jnp
import numpy as np
from jax import lax
from jax.experimental import pallas as pl
from jax.experimental.pallas import tpu as pltpu
from jax.experimental.pallas import tpu_sc as plsc

N = 10000
D = 128
E = 320000
DH = D // 2

NC = 2
NS = 16
NW = NC * NS
NPAD = 10240
EPAD = 327680
K = 128
ROWS_A = EPAD // K // NW
ROWS_C = EPAD // K // NS
RPT = NPAD // NS

_MESH = plsc.VectorSubcoreMesh(
    core_axis_name="c", subcore_axis_name="s", num_cores=NC, num_subcores=NS
)


def _deg_body(ei3_hbm, degp_hbm, packed_hbm, sidx_v, didx_v, ones_v, zrow_v,
              hist_sh, asem, psem):
    c = lax.axis_index("c")
    s = lax.axis_index("s")
    w = c * NS + s
    _ZV = jnp.zeros((16,), jnp.float32)
    _OV = jnp.ones((16,), jnp.float32)

    row0 = pl.multiple_of(w * ROWS_A, 8)
    pltpu.async_copy(ei3_hbm.at[0, pl.ds(row0, ROWS_A)], sidx_v, asem)
    pltpu.async_copy(ei3_hbm.at[1, pl.ds(row0, ROWS_A)], didx_v, asem)

    def fill_o(i, carry):
        ones_v[pl.ds(i * 16, 16)] = _OV
        return carry

    lax.fori_loop(0, K // 16, fill_o, 0)

    def fill_z(i, carry):
        zrow_v[pl.ds(i * 16, 16)] = _ZV
        return carry

    lax.fori_loop(0, RPT // 16, fill_z, 0)

    my_rows = pl.multiple_of(s * RPT, 8)
    pltpu.sync_copy(zrow_v, hist_sh.at[pl.ds(my_rows, RPT)])

    pltpu.make_async_copy(ei3_hbm.at[0, pl.ds(row0, ROWS_A)], sidx_v,
                          asem).wait()
    pltpu.make_async_copy(ei3_hbm.at[1, pl.ds(row0, ROWS_A)], didx_v,
                          asem).wait()
    plsc.subcore_barrier()

    def fire(k, carry):
        pltpu.async_copy(ones_v, hist_sh.at[didx_v.at[k]], asem, add=True)
        return carry

    lax.fori_loop(0, ROWS_A, fire, 0)

    def pack_i(i, carry):
        def pack_j(j, carry2):
            sl = pl.ds(j * 16, 16)
            sidx_v[i, sl] = sidx_v[i, sl] | (didx_v[i, sl] << 16)
            return carry2
        return lax.fori_loop(0, K // 16, pack_j, carry)

    lax.fori_loop(0, ROWS_A, pack_i, 0)
    pltpu.async_copy(sidx_v, packed_hbm.at[pl.ds(row0, ROWS_A)], psem)

    def drain(k, carry):
        pltpu.make_async_copy(ones_v, hist_sh.at[didx_v.at[0]], asem).wait()
        return carry

    lax.fori_loop(0, ROWS_A, drain, 0)
    plsc.subcore_barrier()
    pltpu.sync_copy(hist_sh.at[pl.ds(my_rows, RPT)],
                    degp_hbm.at[c, pl.ds(my_rows, RPT)])
    pltpu.make_async_copy(sidx_v, packed_hbm.at[pl.ds(row0, ROWS_A)],
                          psem).wait()


_deg_kernel = pl.kernel(
    _deg_body,
    out_type=(
        jax.ShapeDtypeStruct((NC, NPAD), jnp.float32),
        jax.ShapeDtypeStruct((EPAD // K, K), jnp.int32),
    ),
    mesh=_MESH,
    scratch_types=[
        pltpu.VMEM((ROWS_A, K), jnp.int32),
        pltpu.VMEM((ROWS_A, K), jnp.int32),
        pltpu.VMEM((K,), jnp.float32),
        pltpu.VMEM((RPT,), jnp.float32),
        pltpu.VMEM_SHARED((NPAD,), jnp.float32),
        pltpu.SemaphoreType.DMA,
        pltpu.SemaphoreType.DMA,
    ],
)


KC = 64
CROWS = EPAD // KC // NW
NBUF = 5
G = CROWS // NBUF

def _agg_body(packed_hbm, y_hbm, aggp_hbm, packed_v, sidx_v, didx_v, rows_v,
              agg_sh, *sems):
    gsem = sems[:NBUF]
    ssem = sems[NBUF:]
    c = lax.axis_index("c")
    s = lax.axis_index("s")
    _ZV = jnp.zeros((16,), jnp.float32)

    PROWS = CROWS * KC // K
    row0 = pl.multiple_of((c * NS + s) * PROWS, 8)
    pltpu.async_copy(packed_hbm.at[pl.ds(row0, PROWS // 2)], packed_v,
                     gsem[0])

    def fz(i, carry):
        def fz2(j, carry2):
            rows_v[0, i, pl.ds(j * 16, 16)] = _ZV
            return carry2
        return lax.fori_loop(0, D // 16, fz2, carry)

    lax.fori_loop(0, KC, fz, 0)

    my_rows = pl.multiple_of(s * RPT, 8)

    def fzd(j, carry):
        off = pl.multiple_of(my_rows + j * KC, 8)
        pltpu.async_copy(rows_v.at[0], agg_sh.at[pl.ds(off, KC)], ssem[0])
        return carry

    lax.fori_loop(0, RPT // KC, fzd, 0)

    pltpu.make_async_copy(packed_hbm.at[pl.ds(row0, PROWS // 2)], packed_v,
                          gsem[0]).wait()

    def fzw(j, carry):
        off = pl.multiple_of(my_rows + j * KC, 8)
        pltpu.make_async_copy(rows_v.at[0], agg_sh.at[pl.ds(off, KC)],
                              ssem[0]).wait()
        return carry

    lax.fori_loop(0, RPT // KC, fzw, 0)
    plsc.subcore_barrier()

    CPR = K // KC

    def unpack(k, b):
        kb = lax.rem(k // CPR, PROWS // 2)
        cb = lax.rem(k, CPR) * KC

        def up(j, carry):
            p = packed_v[kb, pl.ds(cb + j * 16, 16)]
            sidx_v[b, pl.ds(j * 16, 16)] = p & 0xFFFF
            didx_v[b, pl.ds(j * 16, 16)] = lax.shift_right_logical(p, 16)
            return carry
        lax.fori_loop(0, KC // 16, up, 0)

    for b in range(NBUF):
        unpack(b, b)
        pltpu.async_copy(y_hbm.at[sidx_v.at[b]], rows_v.at[b], gsem[b])

    def outer(g, carry):
        @pl.when(g == CROWS // 2 // NBUF - 1)
        def _():
            pltpu.sync_copy(
                packed_hbm.at[pl.ds(row0 + PROWS // 2, PROWS // 2)], packed_v)

        for b in range(NBUF):
            k = g * NBUF + b
            pltpu.make_async_copy(y_hbm.at[sidx_v.at[b]], rows_v.at[b],
                                  gsem[b]).wait()
            pltpu.async_copy(rows_v.at[b], agg_sh.at[didx_v.at[b]], ssem[b],
                             add=True)
            pltpu.make_async_copy(rows_v.at[b], agg_sh.at[didx_v.at[b]],
                                  ssem[b]).wait()
            unpack(k + NBUF, b)
            pltpu.async_copy(y_hbm.at[sidx_v.at[b]], rows_v.at[b], gsem[b])
        return carry

    lax.fori_loop(0, G - 1, outer, 0)
    for b in range(NBUF):
        pltpu.make_async_copy(y_hbm.at[sidx_v.at[b]], rows_v.at[b],
                              gsem[b]).wait()
        pltpu.async_copy(rows_v.at[b], agg_sh.at[didx_v.at[b]], ssem[b],
                         add=True)
    for b in range(NBUF):
        pltpu.make_async_copy(rows_v.at[b], agg_sh.at[didx_v.at[b]],
                              ssem[b]).wait()

    plsc.subcore_barrier()
    pltpu.sync_copy(agg_sh.at[pl.ds(my_rows, RPT)],
                    aggp_hbm.at[c, pl.ds(my_rows, RPT)])


_agg_kernel = pl.kernel(
    _agg_body,
    out_type=jax.ShapeDtypeStruct((NC, NPAD, D), jnp.float32),
    mesh=_MESH,
    scratch_types=[
        pltpu.VMEM((CROWS * KC // K // 2, K), jnp.int32),
        pltpu.VMEM((NBUF, KC), jnp.int32),
        pltpu.VMEM((NBUF, KC), jnp.int32),
        pltpu.VMEM((NBUF, KC, D), jnp.float32),
        pltpu.VMEM_SHARED((NPAD, D), jnp.float32),
    ] + [pltpu.SemaphoreType.DMA] * (2 * NBUF),
)


_BB = 1024

def _y_body(x_ref, w_ref, degp_ref, y_ref):
    xw = jnp.dot(x_ref[...], w_ref[...], preferred_element_type=jnp.float32)
    deg = degp_ref[0, :] + degp_ref[1, :] + 1.0
    dinv = lax.rsqrt(deg)
    y_ref[...] = xw * dinv[:, None]


_y_call = pl.pallas_call(
    _y_body,
    grid=(NPAD // _BB,),
    in_specs=[
        pl.BlockSpec((_BB, D), lambda i: (i, 0)),
        pl.BlockSpec((D, D), lambda i: (0, 0)),
        pl.BlockSpec((NC, _BB), lambda i: (0, i)),
    ],
    out_specs=pl.BlockSpec((_BB, D), lambda i: (i, 0)),
    out_shape=jax.ShapeDtypeStruct((NPAD, D), jnp.float32),
)


_OB = 1024

def _out_body(x_ref, aggp_ref, y_ref, degp_ref, b_ref, o_ref):
    ssum = aggp_ref[0] + aggp_ref[1] + y_ref[...]
    deg = degp_ref[0, :] + degp_ref[1, :] + 1.0
    dinv = lax.rsqrt(deg)
    conv = ssum * dinv[:, None] + b_ref[...]
    o_ref[...] = x_ref[...] + jnp.maximum(conv, 0.0)


_out_call = pl.pallas_call(
    _out_body,
    grid=(pl.cdiv(N, _OB),),
    in_specs=[
        pl.BlockSpec((_OB, D), lambda i: (i, 0)),
        pl.BlockSpec((NC, _OB, D), lambda i: (0, i, 0)),
        pl.BlockSpec((_OB, D), lambda i: (i, 0)),
        pl.BlockSpec((NC, _OB), lambda i: (0, i)),
        pl.BlockSpec((1, D), lambda i: (0, 0)),
    ],
    out_specs=pl.BlockSpec((_OB, D), lambda i: (i, 0)),
    out_shape=jax.ShapeDtypeStruct((N, D), jnp.float32),
)


_PAD = np.broadcast_to(
    (N + (np.arange(EPAD - E, dtype=np.int64) % (NPAD - N))).astype(
        np.int32).reshape(1, (EPAD - E) // K, K),
    (2, (EPAD - E) // K, K)).copy()


def kernel(x, edge_index, W, b):
    ei3 = jnp.concatenate(
        [edge_index.reshape(2, E // K, K), jnp.asarray(_PAD)], axis=1)
    degp, packed = _deg_kernel(ei3)
    y = _y_call(x, W, degp)
    aggp = _agg_kernel(packed, y)
    return _out_call(x, aggp, y, degp, b.reshape(1, D))
